# Initial kernel scaffold; baseline (speedup 1.0000x reference)
import jax
import jax.numpy as jnp
from jax.experimental import pallas as pl


def kernel(node_type, requests, edge_index, active_mask, params):
    raise NotImplementedError("write your pallas kernel here")


# speedup vs baseline: 1.0777x; 1.0777x over previous
import functools

import jax
import jax.numpy as jnp
from jax.experimental import pallas as pl
from jax.experimental.pallas import tpu as pltpu

N = 100000
L = 15
FD = 16
HID = 16
HEADS = 4
DH = HID // HEADS
FC = 128

MLP_B = 1000


def _mlp_body(h_ref, w0_ref, b0_ref, w1_ref, b1_ref, w2_ref, b2_ref,
              w3_ref, b3_ref, wo_ref, bo_ref, out_ref):
    y = jnp.dot(h_ref[...], w0_ref[...], preferred_element_type=jnp.float32)
    y = jax.nn.relu(y + b0_ref[...])
    for w, b in ((w1_ref, b1_ref), (w2_ref, b2_ref), (w3_ref, b3_ref)):
        y = jnp.dot(y, w[...], preferred_element_type=jnp.float32)
        y = jax.nn.relu(y + b[...])
    o = jnp.dot(y, wo_ref[...], preferred_element_type=jnp.float32)
    out_ref[...] = o + bo_ref[...]


def _mlp_logits(h, params):
    w0 = params['W0']
    b0 = params['b0'].reshape(1, FC)
    w1, w2, w3 = params['Wh']
    b1, b2, b3 = [b.reshape(1, FC) for b in params['bh']]
    wo = params['Wo']
    bo = params['bo'].reshape(1, 1)
    full = pl.BlockSpec(lambda i: (0, 0))
    out = pl.pallas_call(
        _mlp_body,
        grid=(N // MLP_B,),
        in_specs=[
            pl.BlockSpec((MLP_B, HID), lambda i: (i, 0)),
            pl.BlockSpec((HID, FC), lambda i: (0, 0)),
            pl.BlockSpec((1, FC), lambda i: (0, 0)),
            pl.BlockSpec((FC, FC), lambda i: (0, 0)),
            pl.BlockSpec((1, FC), lambda i: (0, 0)),
            pl.BlockSpec((FC, FC), lambda i: (0, 0)),
            pl.BlockSpec((1, FC), lambda i: (0, 0)),
            pl.BlockSpec((FC, FC), lambda i: (0, 0)),
            pl.BlockSpec((1, FC), lambda i: (0, 0)),
            pl.BlockSpec((FC, 1), lambda i: (0, 0)),
            pl.BlockSpec((1, 1), lambda i: (0, 0)),
        ],
        out_specs=pl.BlockSpec((MLP_B, 1), lambda i: (i, 0)),
        out_shape=jax.ShapeDtypeStruct((N, 1), jnp.float32),
    )(h, w0, b0, w1, b1, w2, b2, w3, b3, wo, bo)
    return out[:, 0]


def _gat_layer(x, src, dst, p):
    n = x.shape[0]
    xw = (x @ p['W']).reshape(n, HEADS, DH)
    a_src = (xw * p['a_s'][None]).sum(-1)
    a_dst = (xw * p['a_d'][None]).sum(-1)
    e = jax.nn.leaky_relu(a_src[src] + a_dst[dst], 0.2)
    ex = jnp.exp(e)
    s = jax.ops.segment_sum(ex, dst, num_segments=n)
    num = jax.ops.segment_sum(xw[src] * ex[..., None], dst, num_segments=n)
    out = num / (s[..., None] + 1e-30)
    return out.reshape(n, HID) + p['b']


def kernel(node_type, requests, edge_index, active_mask, params):
    x = params['emb'][node_type]
    mean_r = jnp.mean(requests[L:])
    std_r = jnp.std(requests[L:], ddof=1)
    req_final = jnp.concatenate([requests[:L], (requests[L:] - mean_r) / std_r])
    x = jnp.concatenate([x, req_final[:, None]], axis=-1)
    loops = jnp.arange(N)
    src = jnp.concatenate([edge_index[0], loops])
    dst = jnp.concatenate([edge_index[1], loops])
    x = jax.nn.relu(_gat_layer(x, src, dst, params['gat'][0]))
    x = jax.nn.relu(_gat_layer(x, src, dst, params['gat'][1]))
    x = jax.nn.relu(_gat_layer(x, src, dst, params['gat'][2]))
    h = _gat_layer(x, src, dst, params['gat'][3])
    logits1 = _mlp_logits(h, params)
    head = active_mask[:L]
    flipped = jnp.where(head == 0, -jnp.inf, jnp.where(jnp.isneginf(head), 0.0, head))
    remove_mask = jnp.concatenate([flipped, active_mask[L:]])
    removed_logits = logits1 + remove_mask
    g1 = jax.random.gumbel(jax.random.key(1), (N,))
    a1 = jnp.argmax(removed_logits + g1).astype(jnp.int32)
    mask2 = active_mask.at[a1].set(0.0)
    hp = jnp.tanh(h[a1] @ params['Wp'] + params['bp'])
    new_logits = h @ hp + mask2
    g2 = jax.random.gumbel(jax.random.key(2), (N,))
    a2 = jnp.argmax(new_logits + g2).astype(jnp.int32)
    logits = jnp.stack([removed_logits, new_logits])
    actions = jnp.stack([a1, a2])
    return logits, actions


# speedup vs baseline: 55.6447x; 51.6334x over previous
import jax
import jax.numpy as jnp
from jax import lax
from jax.experimental import pallas as pl
from jax.experimental.pallas import tpu as pltpu
from jax.experimental.pallas import tpu_sc as plsc

N = 100000
L = 15
FD = 16
HID = 16
HEADS = 4
DH = HID // HEADS
FC = 128

BN = 1024
NBLK = 98
N_PAD = BN * NBLK
NC, NS = 2, 16
C = 128
E_TOT = 1600000 + N
CHUNKS = -(-E_TOT // (NS * C))
EPT = CHUNKS * C
E_PAD = NS * EPT
ZR = N_PAD // NS

_PERM = [(j % 4) * 4 + j // 4 for j in range(16)]


def _sc_edge_body(src_hbm, dst_hbm, ts_hbm, td_hbm, zeros_hbm, out_hbm,
                  sidx, didx, gs, gd, cb, acc_sh, sem1, sem2):
    c = lax.axis_index("c")
    s = lax.axis_index("s")

    pltpu.sync_copy(zeros_hbm.at[pl.ds(s * ZR, ZR)],
                    acc_sh.at[pl.ds(s * ZR, ZR)])
    plsc.subcore_barrier()

    def chunk(g, carry):
        base = s * EPT + g * C
        pltpu.sync_copy(src_hbm.at[pl.ds(base, C)], sidx)
        pltpu.sync_copy(dst_hbm.at[pl.ds(base, C)], didx)
        pltpu.async_copy(ts_hbm.at[sidx], gs, sem1).wait()
        pltpu.async_copy(td_hbm.at[didx], gd, sem2).wait()

        @pl.when(c == 0)
        def _():
            def edge(r, carry2):
                xwp = gs[r, pl.ds(0, 16)]
                asr = gs[r, pl.ds(16, 16)]
                adr = gd[r, pl.ds(0, 16)]
                e = asr + adr
                e = jnp.where(e >= 0.0, e, 0.2 * e)
                cb[r, pl.ds(0, 16)] = xwp * jnp.exp(e)
                return carry2
            lax.fori_loop(0, C, edge, 0)

        @pl.when(c == 1)
        def _():
            def edge(r, carry2):
                asr = gs[r, pl.ds(16, 16)]
                adr = gd[r, pl.ds(0, 16)]
                e = asr + adr
                e = jnp.where(e >= 0.0, e, 0.2 * e)
                cb[r, pl.ds(0, 16)] = jnp.exp(e)
                return carry2
            lax.fori_loop(0, C, edge, 0)

        pltpu.sync_copy(cb, acc_sh.at[didx], add=True)
        return carry

    lax.fori_loop(0, CHUNKS, chunk, 0)
    plsc.subcore_barrier()
    pltpu.sync_copy(acc_sh.at[pl.ds(s * ZR, ZR)],
                    out_hbm.at[pl.ds(c * N_PAD + s * ZR, ZR)])


def _sc_layer(src, dst, ts, td, zeros):
    mesh = plsc.VectorSubcoreMesh(core_axis_name="c", subcore_axis_name="s")
    f = pl.kernel(
        _sc_edge_body,
        mesh=mesh,
        out_type=jax.ShapeDtypeStruct((2 * N_PAD, 16), jnp.float32),
        scratch_types=[
            pltpu.VMEM((C,), jnp.int32),
            pltpu.VMEM((C,), jnp.int32),
            pltpu.VMEM((C, 32), jnp.float32),
            pltpu.VMEM((C, 16), jnp.float32),
            pltpu.VMEM((C, 16), jnp.float32),
            pltpu.VMEM_SHARED((N_PAD, 16), jnp.float32),
            pltpu.SemaphoreType.DMA,
            pltpu.SemaphoreType.DMA,
        ],
        compiler_params=pltpu.CompilerParams(use_tc_tiling_on_sc=False),
    )
    return f(src, dst, ts, td, zeros)


def _prep0_body(nt_ref, rq_ref, ew_ref, w16_ref, mts_ref, mtd_ref,
                ts_ref, td_ref):
    nt = nt_ref[...]
    sel = (nt == lax.broadcasted_iota(jnp.int32, (BN, 4), 1))
    xw = jnp.dot(sel.astype(jnp.float32), ew_ref[...],
                 preferred_element_type=jnp.float32)
    xw = xw + rq_ref[...] * w16_ref[...]
    ts_ref[...] = jnp.dot(xw, mts_ref[...], preferred_element_type=jnp.float32)
    td_ref[...] = jnp.dot(xw, mtd_ref[...], preferred_element_type=jnp.float32)


def _prep_body(p0_ref, p1_ref, b_ref, w_ref, mts_ref, mtd_ref,
               ts_ref, td_ref):
    z = jax.nn.relu(p0_ref[...] / p1_ref[...] + b_ref[...])
    xw = jnp.dot(z, w_ref[...], preferred_element_type=jnp.float32)
    ts_ref[...] = jnp.dot(xw, mts_ref[...], preferred_element_type=jnp.float32)
    td_ref[...] = jnp.dot(xw, mtd_ref[...], preferred_element_type=jnp.float32)


def _mlp_body(p0_ref, p1_ref, b_ref, pm_ref,
              w0_ref, b0_ref, w1_ref, b1_ref, w2_ref, b2_ref,
              w3_ref, b3_ref, wo_ref, bo_ref, lg_ref, h_ref):
    yp = p0_ref[...] / p1_ref[...] + b_ref[...]
    h = jnp.dot(yp, pm_ref[...], preferred_element_type=jnp.float32)
    h_ref[...] = h
    y = jax.nn.relu(jnp.dot(h, w0_ref[...],
                            preferred_element_type=jnp.float32) + b0_ref[...])
    for w, b in ((w1_ref, b1_ref), (w2_ref, b2_ref), (w3_ref, b3_ref)):
        y = jax.nn.relu(jnp.dot(y, w[...],
                                preferred_element_type=jnp.float32) + b[...])
    lg_ref[...] = jnp.dot(y, wo_ref[...],
                          preferred_element_type=jnp.float32) + bo_ref[...]


def _matvec_body(h_ref, hp_ref, m_ref, o_ref):
    o_ref[...] = jnp.dot(h_ref[...], hp_ref[...],
                         preferred_element_type=jnp.float32) + m_ref[...]


def _row_spec(w):
    return pl.BlockSpec((BN, w), lambda i: (i, 0))


def _full_spec(a, b):
    return pl.BlockSpec((a, b), lambda i: (0, 0))


def _prep0(nt, rq, ew, w16, mts, mtd):
    return pl.pallas_call(
        _prep0_body, grid=(NBLK,),
        in_specs=[_row_spec(1), _row_spec(1), _full_spec(4, 16),
                  _full_spec(1, 16), _full_spec(16, 32), _full_spec(16, 16)],
        out_specs=[_row_spec(32), _row_spec(16)],
        out_shape=[jax.ShapeDtypeStruct((N_PAD, 32), jnp.float32),
                   jax.ShapeDtypeStruct((N_PAD, 16), jnp.float32)],
    )(nt, rq, ew, w16, mts, mtd)


def _prep(p0, p1, b, w, mts, mtd):
    return pl.pallas_call(
        _prep_body, grid=(NBLK,),
        in_specs=[_row_spec(16), _row_spec(16),
                  _full_spec(1, 16), _full_spec(16, 16),
                  _full_spec(16, 32), _full_spec(16, 16)],
        out_specs=[_row_spec(32), _row_spec(16)],
        out_shape=[jax.ShapeDtypeStruct((N_PAD, 32), jnp.float32),
                   jax.ShapeDtypeStruct((N_PAD, 16), jnp.float32)],
    )(p0, p1, b, w, mts, mtd)


def _mlp(p0, p1, bp, pm, params):
    args = [p0, p1, bp, pm, params['W0'], params['b0'].reshape(1, FC)]
    for w, b in zip(params['Wh'], params['bh']):
        args += [w, b.reshape(1, FC)]
    args += [params['Wo'], params['bo'].reshape(1, 1)]
    return pl.pallas_call(
        _mlp_body, grid=(NBLK,),
        in_specs=[_row_spec(16), _row_spec(16),
                  _full_spec(1, 16), _full_spec(16, 16),
                  _full_spec(HID, FC), _full_spec(1, FC),
                  _full_spec(FC, FC), _full_spec(1, FC),
                  _full_spec(FC, FC), _full_spec(1, FC),
                  _full_spec(FC, FC), _full_spec(1, FC),
                  _full_spec(FC, 1), _full_spec(1, 1)],
        out_specs=[_row_spec(1), _row_spec(16)],
        out_shape=[jax.ShapeDtypeStruct((N_PAD, 1), jnp.float32),
                   jax.ShapeDtypeStruct((N_PAD, 16), jnp.float32)],
    )(*args)


def _matvec(h, hp, m):
    return pl.pallas_call(
        _matvec_body, grid=(NBLK,),
        in_specs=[_row_spec(16), _full_spec(16, 1), _row_spec(1)],
        out_specs=_row_spec(1),
        out_shape=jax.ShapeDtypeStruct((N_PAD, 1), jnp.float32),
    )(h, hp, m)


def _fold(a):
    s = jnp.zeros((HID, HEADS), jnp.float32)
    for h in range(HEADS):
        s = s.at[h * DH:(h + 1) * DH, h].set(a[h])
    return s


def kernel(node_type, requests, edge_index, active_mask, params):
    mean_r = jnp.mean(requests[L:])
    std_r = jnp.std(requests[L:], ddof=1)
    req_final = jnp.concatenate([requests[:L], (requests[L:] - mean_r) / std_r])
    rq = jnp.pad(req_final, (0, N_PAD - N)).reshape(N_PAD, 1)
    nt = jnp.pad(node_type.astype(jnp.int32), (0, N_PAD - N)).reshape(N_PAD, 1)

    loops = jnp.arange(N, dtype=edge_index.dtype)
    src = jnp.concatenate([edge_index[0], loops]).astype(jnp.int32)
    dst = jnp.concatenate([edge_index[1], loops]).astype(jnp.int32)
    src = jnp.pad(src, (0, E_PAD - E_TOT), constant_values=N)
    dst = jnp.pad(dst, (0, E_PAD - E_TOT), constant_values=N)

    gat = params['gat']
    pm = jnp.eye(HID, dtype=jnp.float32)[:, _PERM]
    trep = jnp.tile(jnp.eye(HEADS, dtype=jnp.float32), (1, 4))
    mts = [jnp.concatenate([pm, _fold(g['a_s']) @ trep], axis=1) for g in gat]
    mtd = [_fold(g['a_d']) @ trep for g in gat]
    weff = [None] + [pm @ g['W'] for g in gat[1:]]
    bperm = [g['b'] @ pm for g in gat]
    ew0 = params['emb'] @ gat[0]['W'][:FD]
    w16 = gat[0]['W'][FD].reshape(1, HID)
    zeros = jnp.zeros((N_PAD, 16), jnp.float32)

    ts, td = _prep0(nt, rq, ew0, w16, mts[0], mtd[0])

    for li in range(4):
        part = _sc_layer(src, dst, ts, td, zeros)
        p0 = part[:N_PAD]
        p1 = part[N_PAD:]
        if li < 3:
            ts, td = _prep(p0, p1, bperm[li].reshape(1, HID),
                           weff[li + 1], mts[li + 1], mtd[li + 1])

    lg, h = _mlp(p0, p1, bperm[3].reshape(1, HID), pm, params)
    logits1 = lg[:N, 0]
    h = h[:N]

    head = active_mask[:L]
    flipped = jnp.where(head == 0, -jnp.inf,
                        jnp.where(jnp.isneginf(head), 0.0, head))
    remove_mask = jnp.concatenate([flipped, active_mask[L:]])
    removed_logits = logits1 + remove_mask
    g1 = jax.random.gumbel(jax.random.key(1), (N,))
    a1 = jnp.argmax(removed_logits + g1).astype(jnp.int32)
    mask2 = active_mask.at[a1].set(0.0)
    hp = jnp.tanh(h[a1] @ params['Wp'] + params['bp']).reshape(HID, 1)
    m2 = jnp.pad(mask2, (0, N_PAD - N)).reshape(N_PAD, 1)
    new_logits = _matvec(jnp.pad(h, ((0, N_PAD - N), (0, 0))), hp, m2)[:N, 0]
    g2 = jax.random.gumbel(jax.random.key(2), (N,))
    a2 = jnp.argmax(new_logits + g2).astype(jnp.int32)
    logits = jnp.stack([removed_logits, new_logits])
    actions = jnp.stack([a1, a2])
    return logits, actions


# speedup vs baseline: 102.8843x; 1.8489x over previous
import jax
import jax.numpy as jnp
from jax import lax
from jax.experimental import pallas as pl
from jax.experimental.pallas import tpu as pltpu
from jax.experimental.pallas import tpu_sc as plsc

N = 100000
L = 15
FD = 16
HID = 16
HEADS = 4
DH = HID // HEADS
FC = 128

BN = 1024
NBLK = 98
N_PAD = BN * NBLK
NC, NS = 2, 16
C = 128
K = 8
E_TOT = 1600000 + N
CHUNKS = K * (-(-E_TOT // (NS * C * K)))
EPT = CHUNKS * C
E_PAD = NS * EPT
TOT_CH = NS * CHUNKS
ZR = N_PAD // NS

_PERM = [(j % 4) * 4 + j // 4 for j in range(16)]


def _sc_edge_body(src_hbm, dst_hbm, ts_hbm, td_hbm, zeros_hbm, out_hbm,
                  sidxb, didxb, gs0, gs1, gd0, gd1, cb0, cb1, acc_sh,
                  sgs0, sgs1, sgd0, sgd1, ss0, ss1):
    c = lax.axis_index("c")
    s = lax.axis_index("s")
    gs = (gs0, gs1)
    gd = (gd0, gd1)
    cb = (cb0, cb1)
    sgs = (sgs0, sgs1)
    sgd = (sgd0, sgd1)
    ss = (ss0, ss1)
    is_num = c == 0

    pltpu.sync_copy(zeros_hbm.at[pl.ds(s * ZR, ZR)],
                    acc_sh.at[pl.ds(s * ZR, ZR)])
    plsc.subcore_barrier()

    def block(b, carry):
        row0 = s * CHUNKS + b * K
        pltpu.sync_copy(src_hbm.at[pl.ds(row0, K)], sidxb)
        pltpu.sync_copy(dst_hbm.at[pl.ds(row0, K)], didxb)

        hg = [None, None]
        hd = [None, None]
        hs = [None, None]
        hg[0] = pltpu.async_copy(ts_hbm.at[sidxb.at[0]], gs[0], sgs[0])
        hd[0] = pltpu.async_copy(td_hbm.at[didxb.at[0]], gd[0], sgd[0])
        for k in range(K):
            sl = k % 2
            ns_ = (k + 1) % 2
            if k < K - 1:
                hg[ns_] = pltpu.async_copy(ts_hbm.at[sidxb.at[k + 1]],
                                           gs[ns_], sgs[ns_])
                hd[ns_] = pltpu.async_copy(td_hbm.at[didxb.at[k + 1]],
                                           gd[ns_], sgd[ns_])
            hg[sl].wait()
            hd[sl].wait()
            if k >= 2:
                hs[sl].wait()

            gsl = gs[sl]
            gdl = gd[sl]
            cbl = cb[sl]

            def edge4(i, carry2):
                for u in range(4):
                    r = i * 4 + u
                    xwp = gsl[r, pl.ds(0, 16)]
                    asr = gsl[r, pl.ds(16, 16)]
                    adr = gdl[r, pl.ds(0, 16)]
                    e = asr + adr
                    e = jnp.where(e >= 0.0, e, 0.2 * e)
                    fac = jnp.where(is_num, xwp, 1.0)
                    cbl[r, pl.ds(0, 16)] = fac * jnp.exp(e)
                return carry2

            lax.fori_loop(0, C // 4, edge4, 0)
            hs[sl] = pltpu.async_copy(cbl, acc_sh.at[didxb.at[k]], ss[sl],
                                      add=True)
        hs[0].wait()
        hs[1].wait()
        return carry

    lax.fori_loop(0, CHUNKS // K, block, 0)
    plsc.subcore_barrier()
    pltpu.sync_copy(acc_sh.at[pl.ds(s * ZR, ZR)],
                    out_hbm.at[pl.ds(c * N_PAD + s * ZR, ZR)])


def _sc_layer(src, dst, ts, td, zeros):
    mesh = plsc.VectorSubcoreMesh(core_axis_name="c", subcore_axis_name="s")
    f = pl.kernel(
        _sc_edge_body,
        mesh=mesh,
        out_type=jax.ShapeDtypeStruct((2 * N_PAD, 16), jnp.float32),
        scratch_types=[
            pltpu.VMEM((K, C), jnp.int32),
            pltpu.VMEM((K, C), jnp.int32),
            pltpu.VMEM((C, 32), jnp.float32),
            pltpu.VMEM((C, 32), jnp.float32),
            pltpu.VMEM((C, 16), jnp.float32),
            pltpu.VMEM((C, 16), jnp.float32),
            pltpu.VMEM((C, 16), jnp.float32),
            pltpu.VMEM((C, 16), jnp.float32),
            pltpu.VMEM_SHARED((N_PAD, 16), jnp.float32),
            pltpu.SemaphoreType.DMA,
            pltpu.SemaphoreType.DMA,
            pltpu.SemaphoreType.DMA,
            pltpu.SemaphoreType.DMA,
            pltpu.SemaphoreType.DMA,
            pltpu.SemaphoreType.DMA,
        ],
        compiler_params=pltpu.CompilerParams(use_tc_tiling_on_sc=False),
    )
    return f(src, dst, ts, td, zeros)


def _prep0_body(nt_ref, rq_ref, ew_ref, w16_ref, mts_ref, mtd_ref,
                ts_ref, td_ref):
    nt = nt_ref[...]
    sel = (nt == lax.broadcasted_iota(jnp.int32, (BN, 4), 1))
    xw = jnp.dot(sel.astype(jnp.float32), ew_ref[...],
                 preferred_element_type=jnp.float32)
    xw = xw + rq_ref[...] * w16_ref[...]
    ts_ref[...] = jnp.dot(xw, mts_ref[...], preferred_element_type=jnp.float32)
    td_ref[...] = jnp.dot(xw, mtd_ref[...], preferred_element_type=jnp.float32)


def _prep_body(p0_ref, p1_ref, b_ref, w_ref, mts_ref, mtd_ref,
               ts_ref, td_ref):
    z = jax.nn.relu(p0_ref[...] / p1_ref[...] + b_ref[...])
    xw = jnp.dot(z, w_ref[...], preferred_element_type=jnp.float32)
    ts_ref[...] = jnp.dot(xw, mts_ref[...], preferred_element_type=jnp.float32)
    td_ref[...] = jnp.dot(xw, mtd_ref[...], preferred_element_type=jnp.float32)


def _mlp_body(p0_ref, p1_ref, b_ref, pm_ref,
              w0_ref, b0_ref, w1_ref, b1_ref, w2_ref, b2_ref,
              w3_ref, b3_ref, wo_ref, bo_ref, lg_ref, h_ref):
    yp = p0_ref[...] / p1_ref[...] + b_ref[...]
    h = jnp.dot(yp, pm_ref[...], preferred_element_type=jnp.float32)
    h_ref[...] = h
    y = jax.nn.relu(jnp.dot(h, w0_ref[...],
                            preferred_element_type=jnp.float32) + b0_ref[...])
    for w, b in ((w1_ref, b1_ref), (w2_ref, b2_ref), (w3_ref, b3_ref)):
        y = jax.nn.relu(jnp.dot(y, w[...],
                                preferred_element_type=jnp.float32) + b[...])
    lg_ref[...] = jnp.dot(y, wo_ref[...],
                          preferred_element_type=jnp.float32) + bo_ref[...]


def _matvec_body(h_ref, hp_ref, m_ref, o_ref):
    o_ref[...] = jnp.dot(h_ref[...], hp_ref[...],
                         preferred_element_type=jnp.float32) + m_ref[...]


def _row_spec(w):
    return pl.BlockSpec((BN, w), lambda i: (i, 0))


def _full_spec(a, b):
    return pl.BlockSpec((a, b), lambda i: (0, 0))


def _prep0(nt, rq, ew, w16, mts, mtd):
    return pl.pallas_call(
        _prep0_body, grid=(NBLK,),
        in_specs=[_row_spec(1), _row_spec(1), _full_spec(4, 16),
                  _full_spec(1, 16), _full_spec(16, 32), _full_spec(16, 16)],
        out_specs=[_row_spec(32), _row_spec(16)],
        out_shape=[jax.ShapeDtypeStruct((N_PAD, 32), jnp.float32),
                   jax.ShapeDtypeStruct((N_PAD, 16), jnp.float32)],
    )(nt, rq, ew, w16, mts, mtd)


def _prep(p0, p1, b, w, mts, mtd):
    return pl.pallas_call(
        _prep_body, grid=(NBLK,),
        in_specs=[_row_spec(16), _row_spec(16),
                  _full_spec(1, 16), _full_spec(16, 16),
                  _full_spec(16, 32), _full_spec(16, 16)],
        out_specs=[_row_spec(32), _row_spec(16)],
        out_shape=[jax.ShapeDtypeStruct((N_PAD, 32), jnp.float32),
                   jax.ShapeDtypeStruct((N_PAD, 16), jnp.float32)],
    )(p0, p1, b, w, mts, mtd)


def _mlp(p0, p1, bp, pm, params):
    args = [p0, p1, bp, pm, params['W0'], params['b0'].reshape(1, FC)]
    for w, b in zip(params['Wh'], params['bh']):
        args += [w, b.reshape(1, FC)]
    args += [params['Wo'], params['bo'].reshape(1, 1)]
    return pl.pallas_call(
        _mlp_body, grid=(NBLK,),
        in_specs=[_row_spec(16), _row_spec(16),
                  _full_spec(1, 16), _full_spec(16, 16),
                  _full_spec(HID, FC), _full_spec(1, FC),
                  _full_spec(FC, FC), _full_spec(1, FC),
                  _full_spec(FC, FC), _full_spec(1, FC),
                  _full_spec(FC, FC), _full_spec(1, FC),
                  _full_spec(FC, 1), _full_spec(1, 1)],
        out_specs=[_row_spec(1), _row_spec(16)],
        out_shape=[jax.ShapeDtypeStruct((N_PAD, 1), jnp.float32),
                   jax.ShapeDtypeStruct((N_PAD, 16), jnp.float32)],
    )(*args)


def _matvec(h, hp, m):
    return pl.pallas_call(
        _matvec_body, grid=(NBLK,),
        in_specs=[_row_spec(16), _full_spec(16, 1), _row_spec(1)],
        out_specs=_row_spec(1),
        out_shape=jax.ShapeDtypeStruct((N_PAD, 1), jnp.float32),
    )(h, hp, m)


def _fold(a):
    s = jnp.zeros((HID, HEADS), jnp.float32)
    for h in range(HEADS):
        s = s.at[h * DH:(h + 1) * DH, h].set(a[h])
    return s


def kernel(node_type, requests, edge_index, active_mask, params):
    mean_r = jnp.mean(requests[L:])
    std_r = jnp.std(requests[L:], ddof=1)
    req_final = jnp.concatenate([requests[:L], (requests[L:] - mean_r) / std_r])
    rq = jnp.pad(req_final, (0, N_PAD - N)).reshape(N_PAD, 1)
    nt = jnp.pad(node_type.astype(jnp.int32), (0, N_PAD - N)).reshape(N_PAD, 1)

    loops = jnp.arange(N, dtype=edge_index.dtype)
    src = jnp.concatenate([edge_index[0], loops]).astype(jnp.int32)
    dst = jnp.concatenate([edge_index[1], loops]).astype(jnp.int32)
    src = jnp.pad(src, (0, E_PAD - E_TOT), constant_values=N).reshape(TOT_CH, C)
    dst = jnp.pad(dst, (0, E_PAD - E_TOT), constant_values=N).reshape(TOT_CH, C)

    gat = params['gat']
    pm = jnp.eye(HID, dtype=jnp.float32)[:, _PERM]
    trep = jnp.tile(jnp.eye(HEADS, dtype=jnp.float32), (1, 4))
    mts = [jnp.concatenate([pm, _fold(g['a_s']) @ trep], axis=1) for g in gat]
    mtd = [_fold(g['a_d']) @ trep for g in gat]
    weff = [None] + [pm @ g['W'] for g in gat[1:]]
    bperm = [g['b'] @ pm for g in gat]
    ew0 = params['emb'] @ gat[0]['W'][:FD]
    w16 = gat[0]['W'][FD].reshape(1, HID)
    zeros = jnp.zeros((N_PAD, 16), jnp.float32)

    ts, td = _prep0(nt, rq, ew0, w16, mts[0], mtd[0])

    for li in range(4):
        part = _sc_layer(src, dst, ts, td, zeros)
        p0 = part[:N_PAD]
        p1 = part[N_PAD:]
        if li < 3:
            ts, td = _prep(p0, p1, bperm[li].reshape(1, HID),
                           weff[li + 1], mts[li + 1], mtd[li + 1])

    lg, h = _mlp(p0, p1, bperm[3].reshape(1, HID), pm, params)
    logits1 = lg[:N, 0]
    h = h[:N]

    head = active_mask[:L]
    flipped = jnp.where(head == 0, -jnp.inf,
                        jnp.where(jnp.isneginf(head), 0.0, head))
    remove_mask = jnp.concatenate([flipped, active_mask[L:]])
    removed_logits = logits1 + remove_mask
    g1 = jax.random.gumbel(jax.random.key(1), (N,))
    a1 = jnp.argmax(removed_logits + g1).astype(jnp.int32)
    mask2 = active_mask.at[a1].set(0.0)
    hp = jnp.tanh(h[a1] @ params['Wp'] + params['bp']).reshape(HID, 1)
    m2 = jnp.pad(mask2, (0, N_PAD - N)).reshape(N_PAD, 1)
    new_logits = _matvec(jnp.pad(h, ((0, N_PAD - N), (0, 0))), hp, m2)[:N, 0]
    g2 = jax.random.gumbel(jax.random.key(2), (N,))
    a2 = jnp.argmax(new_logits + g2).astype(jnp.int32)
    logits = jnp.stack([removed_logits, new_logits])
    actions = jnp.stack([a1, a2])
    return logits, actions


# speedup vs baseline: 202.9072x; 1.9722x over previous
import jax
import jax.numpy as jnp
from jax import lax
from jax.experimental import pallas as pl
from jax.experimental.pallas import tpu as pltpu
from jax.experimental.pallas import tpu_sc as plsc

N = 100000
L = 15
FD = 16
HID = 16
HEADS = 4
DH = HID // HEADS
FC = 128

BN = 1024
NBLK = 98
N_PAD = BN * NBLK
NC, NS = 2, 16
C = 128
K = 8
E_TOT = 1600000 + N
CHUNKS = K * (-(-E_TOT // (NS * C * K)))
EPT = CHUNKS * C
E_PAD = NS * EPT
TOT_CH = NS * CHUNKS
ZR = N_PAD // NS

_PERM = [(j % 4) * 4 + j // 4 for j in range(16)]


def _sc_edge_body(src_hbm, dst_hbm, ts_hbm, td_hbm, zeros_hbm, out_hbm,
                  sidxb, didxb, gs0, gs1, gd0, gd1, cb0, cb1, acc_sh,
                  sgs0, sgs1, sgd0, sgd1, ss0, ss1):
    c = lax.axis_index("c")
    s = lax.axis_index("s")
    gs = (gs0, gs1)
    gd = (gd0, gd1)
    cb = (cb0, cb1)
    sgs = (sgs0, sgs1)
    sgd = (sgd0, sgd1)
    ss = (ss0, ss1)
    is_num = c == 0

    pltpu.sync_copy(zeros_hbm.at[pl.ds(s * ZR, ZR)],
                    acc_sh.at[pl.ds(s * ZR, ZR)])
    plsc.subcore_barrier()

    def block(b, carry):
        row0 = s * CHUNKS + b * K
        pltpu.sync_copy(src_hbm.at[pl.ds(row0, K)], sidxb)
        pltpu.sync_copy(dst_hbm.at[pl.ds(row0, K)], didxb)

        hg = [None, None]
        hd = [None, None]
        hs = [None, None]
        hg[0] = pltpu.async_copy(ts_hbm.at[sidxb.at[0]], gs[0], sgs[0])
        hd[0] = pltpu.async_copy(td_hbm.at[didxb.at[0]], gd[0], sgd[0])
        for k in range(K):
            sl = k % 2
            ns_ = (k + 1) % 2
            if k < K - 1:
                hg[ns_] = pltpu.async_copy(ts_hbm.at[sidxb.at[k + 1]],
                                           gs[ns_], sgs[ns_])
                hd[ns_] = pltpu.async_copy(td_hbm.at[didxb.at[k + 1]],
                                           gd[ns_], sgd[ns_])
            hg[sl].wait()
            hd[sl].wait()
            if k >= 2:
                hs[sl].wait()

            gsl = gs[sl]
            gdl = gd[sl]
            cbl = cb[sl]

            @pl.when(is_num)
            def _():
                @plsc.parallel_loop(0, C, step=1, unroll=8)
                def _num(r):
                    xwp = gsl[r, pl.ds(0, 16)]
                    asr = gsl[r, pl.ds(16, 16)]
                    adr = gdl[r, pl.ds(0, 16)]
                    e = asr + adr
                    e = jnp.where(e >= 0.0, e, 0.2 * e)
                    cbl[r, pl.ds(0, 16)] = xwp * jnp.exp(e)

            @pl.when(jnp.logical_not(is_num))
            def _():
                @plsc.parallel_loop(0, C, step=1, unroll=8)
                def _den(r):
                    asr = gsl[r, pl.ds(16, 16)]
                    adr = gdl[r, pl.ds(0, 16)]
                    e = asr + adr
                    e = jnp.where(e >= 0.0, e, 0.2 * e)
                    cbl[r, pl.ds(0, 16)] = jnp.exp(e)
            hs[sl] = pltpu.async_copy(cbl, acc_sh.at[didxb.at[k]], ss[sl],
                                      add=True)
        hs[0].wait()
        hs[1].wait()
        return carry

    lax.fori_loop(0, CHUNKS // K, block, 0)
    plsc.subcore_barrier()
    pltpu.sync_copy(acc_sh.at[pl.ds(s * ZR, ZR)],
                    out_hbm.at[pl.ds(c * N_PAD + s * ZR, ZR)])


def _sc_layer(src, dst, ts, td, zeros):
    mesh = plsc.VectorSubcoreMesh(core_axis_name="c", subcore_axis_name="s")
    f = pl.kernel(
        _sc_edge_body,
        mesh=mesh,
        out_type=jax.ShapeDtypeStruct((2 * N_PAD, 16), jnp.float32),
        scratch_types=[
            pltpu.VMEM((K, C), jnp.int32),
            pltpu.VMEM((K, C), jnp.int32),
            pltpu.VMEM((C, 32), jnp.float32),
            pltpu.VMEM((C, 32), jnp.float32),
            pltpu.VMEM((C, 16), jnp.float32),
            pltpu.VMEM((C, 16), jnp.float32),
            pltpu.VMEM((C, 16), jnp.float32),
            pltpu.VMEM((C, 16), jnp.float32),
            pltpu.VMEM_SHARED((N_PAD, 16), jnp.float32),
            pltpu.SemaphoreType.DMA,
            pltpu.SemaphoreType.DMA,
            pltpu.SemaphoreType.DMA,
            pltpu.SemaphoreType.DMA,
            pltpu.SemaphoreType.DMA,
            pltpu.SemaphoreType.DMA,
        ],
        compiler_params=pltpu.CompilerParams(use_tc_tiling_on_sc=False),
    )
    return f(src, dst, ts, td, zeros)


def _prep0_body(nt_ref, rq_ref, ew_ref, w16_ref, mts_ref, mtd_ref,
                ts_ref, td_ref):
    nt = nt_ref[...]
    sel = (nt == lax.broadcasted_iota(jnp.int32, (BN, 4), 1))
    xw = jnp.dot(sel.astype(jnp.float32), ew_ref[...],
                 preferred_element_type=jnp.float32)
    xw = xw + rq_ref[...] * w16_ref[...]
    ts_ref[...] = jnp.dot(xw, mts_ref[...], preferred_element_type=jnp.float32)
    td_ref[...] = jnp.dot(xw, mtd_ref[...], preferred_element_type=jnp.float32)


def _prep_body(p0_ref, p1_ref, b_ref, w_ref, mts_ref, mtd_ref,
               ts_ref, td_ref):
    z = jax.nn.relu(p0_ref[...] / p1_ref[...] + b_ref[...])
    xw = jnp.dot(z, w_ref[...], preferred_element_type=jnp.float32)
    ts_ref[...] = jnp.dot(xw, mts_ref[...], preferred_element_type=jnp.float32)
    td_ref[...] = jnp.dot(xw, mtd_ref[...], preferred_element_type=jnp.float32)


def _mlp_body(p0_ref, p1_ref, b_ref, pm_ref,
              w0_ref, b0_ref, w1_ref, b1_ref, w2_ref, b2_ref,
              w3_ref, b3_ref, wo_ref, bo_ref, lg_ref, h_ref):
    yp = p0_ref[...] / p1_ref[...] + b_ref[...]
    h = jnp.dot(yp, pm_ref[...], preferred_element_type=jnp.float32)
    h_ref[...] = h
    y = jax.nn.relu(jnp.dot(h, w0_ref[...],
                            preferred_element_type=jnp.float32) + b0_ref[...])
    for w, b in ((w1_ref, b1_ref), (w2_ref, b2_ref), (w3_ref, b3_ref)):
        y = jax.nn.relu(jnp.dot(y, w[...],
                                preferred_element_type=jnp.float32) + b[...])
    lg_ref[...] = jnp.dot(y, wo_ref[...],
                          preferred_element_type=jnp.float32) + bo_ref[...]


def _matvec_body(h_ref, hp_ref, m_ref, o_ref):
    o_ref[...] = jnp.dot(h_ref[...], hp_ref[...],
                         preferred_element_type=jnp.float32) + m_ref[...]


def _row_spec(w):
    return pl.BlockSpec((BN, w), lambda i: (i, 0))


def _full_spec(a, b):
    return pl.BlockSpec((a, b), lambda i: (0, 0))


def _prep0(nt, rq, ew, w16, mts, mtd):
    return pl.pallas_call(
        _prep0_body, grid=(NBLK,),
        in_specs=[_row_spec(1), _row_spec(1), _full_spec(4, 16),
                  _full_spec(1, 16), _full_spec(16, 32), _full_spec(16, 16)],
        out_specs=[_row_spec(32), _row_spec(16)],
        out_shape=[jax.ShapeDtypeStruct((N_PAD, 32), jnp.float32),
                   jax.ShapeDtypeStruct((N_PAD, 16), jnp.float32)],
    )(nt, rq, ew, w16, mts, mtd)


def _prep(p0, p1, b, w, mts, mtd):
    return pl.pallas_call(
        _prep_body, grid=(NBLK,),
        in_specs=[_row_spec(16), _row_spec(16),
                  _full_spec(1, 16), _full_spec(16, 16),
                  _full_spec(16, 32), _full_spec(16, 16)],
        out_specs=[_row_spec(32), _row_spec(16)],
        out_shape=[jax.ShapeDtypeStruct((N_PAD, 32), jnp.float32),
                   jax.ShapeDtypeStruct((N_PAD, 16), jnp.float32)],
    )(p0, p1, b, w, mts, mtd)


def _mlp(p0, p1, bp, pm, params):
    args = [p0, p1, bp, pm, params['W0'], params['b0'].reshape(1, FC)]
    for w, b in zip(params['Wh'], params['bh']):
        args += [w, b.reshape(1, FC)]
    args += [params['Wo'], params['bo'].reshape(1, 1)]
    return pl.pallas_call(
        _mlp_body, grid=(NBLK,),
        in_specs=[_row_spec(16), _row_spec(16),
                  _full_spec(1, 16), _full_spec(16, 16),
                  _full_spec(HID, FC), _full_spec(1, FC),
                  _full_spec(FC, FC), _full_spec(1, FC),
                  _full_spec(FC, FC), _full_spec(1, FC),
                  _full_spec(FC, FC), _full_spec(1, FC),
                  _full_spec(FC, 1), _full_spec(1, 1)],
        out_specs=[_row_spec(1), _row_spec(16)],
        out_shape=[jax.ShapeDtypeStruct((N_PAD, 1), jnp.float32),
                   jax.ShapeDtypeStruct((N_PAD, 16), jnp.float32)],
    )(*args)


def _matvec(h, hp, m):
    return pl.pallas_call(
        _matvec_body, grid=(NBLK,),
        in_specs=[_row_spec(16), _full_spec(16, 1), _row_spec(1)],
        out_specs=_row_spec(1),
        out_shape=jax.ShapeDtypeStruct((N_PAD, 1), jnp.float32),
    )(h, hp, m)


def _fold(a):
    s = jnp.zeros((HID, HEADS), jnp.float32)
    for h in range(HEADS):
        s = s.at[h * DH:(h + 1) * DH, h].set(a[h])
    return s


def kernel(node_type, requests, edge_index, active_mask, params):
    mean_r = jnp.mean(requests[L:])
    std_r = jnp.std(requests[L:], ddof=1)
    req_final = jnp.concatenate([requests[:L], (requests[L:] - mean_r) / std_r])
    rq = jnp.pad(req_final, (0, N_PAD - N)).reshape(N_PAD, 1)
    nt = jnp.pad(node_type.astype(jnp.int32), (0, N_PAD - N)).reshape(N_PAD, 1)

    loops = jnp.arange(N, dtype=edge_index.dtype)
    src = jnp.concatenate([edge_index[0], loops]).astype(jnp.int32)
    dst = jnp.concatenate([edge_index[1], loops]).astype(jnp.int32)
    src = jnp.pad(src, (0, E_PAD - E_TOT), constant_values=N).reshape(TOT_CH, C)
    dst = jnp.pad(dst, (0, E_PAD - E_TOT), constant_values=N).reshape(TOT_CH, C)

    gat = params['gat']
    pm = jnp.eye(HID, dtype=jnp.float32)[:, _PERM]
    trep = jnp.tile(jnp.eye(HEADS, dtype=jnp.float32), (1, 4))
    mts = [jnp.concatenate([pm, _fold(g['a_s']) @ trep], axis=1) for g in gat]
    mtd = [_fold(g['a_d']) @ trep for g in gat]
    weff = [None] + [pm @ g['W'] for g in gat[1:]]
    bperm = [g['b'] @ pm for g in gat]
    ew0 = params['emb'] @ gat[0]['W'][:FD]
    w16 = gat[0]['W'][FD].reshape(1, HID)
    zeros = jnp.zeros((N_PAD, 16), jnp.float32)

    ts, td = _prep0(nt, rq, ew0, w16, mts[0], mtd[0])

    for li in range(4):
        part = _sc_layer(src, dst, ts, td, zeros)
        p0 = part[:N_PAD]
        p1 = part[N_PAD:]
        if li < 3:
            ts, td = _prep(p0, p1, bperm[li].reshape(1, HID),
                           weff[li + 1], mts[li + 1], mtd[li + 1])

    lg, h = _mlp(p0, p1, bperm[3].reshape(1, HID), pm, params)
    logits1 = lg[:N, 0]
    h = h[:N]

    head = active_mask[:L]
    flipped = jnp.where(head == 0, -jnp.inf,
                        jnp.where(jnp.isneginf(head), 0.0, head))
    remove_mask = jnp.concatenate([flipped, active_mask[L:]])
    removed_logits = logits1 + remove_mask
    g1 = jax.random.gumbel(jax.random.key(1), (N,))
    a1 = jnp.argmax(removed_logits + g1).astype(jnp.int32)
    mask2 = active_mask.at[a1].set(0.0)
    hp = jnp.tanh(h[a1] @ params['Wp'] + params['bp']).reshape(HID, 1)
    m2 = jnp.pad(mask2, (0, N_PAD - N)).reshape(N_PAD, 1)
    new_logits = _matvec(jnp.pad(h, ((0, N_PAD - N), (0, 0))), hp, m2)[:N, 0]
    g2 = jax.random.gumbel(jax.random.key(2), (N,))
    a2 = jnp.argmax(new_logits + g2).astype(jnp.int32)
    logits = jnp.stack([removed_logits, new_logits])
    actions = jnp.stack([a1, a2])
    return logits, actions


# speedup vs baseline: 214.3697x; 1.0565x over previous
import jax
import jax.numpy as jnp
from jax import lax
from jax.experimental import pallas as pl
from jax.experimental.pallas import tpu as pltpu
from jax.experimental.pallas import tpu_sc as plsc

N = 100000
L = 15
FD = 16
HID = 16
HEADS = 4
DH = HID // HEADS
FC = 128

BN = 1024
NBLK = 98
N_PAD = BN * NBLK
NC, NS = 2, 16
C = 128
K = 8
E_TOT = 1600000 + N
CHUNKS = K * (-(-E_TOT // (NS * C * K)))
EPT = CHUNKS * C
E_PAD = NS * EPT
TOT_CH = NS * CHUNKS
ZR = N_PAD // NS

_PERM = [(j % 4) * 4 + j // 4 for j in range(16)]


def _sc_edge_body(src_hbm, dst_hbm, ts_hbm, td_hbm, zeros_hbm, out_hbm,
                  sidxb, didxb, gs0, gs1, gs2, gd0, gd1, gd2, cb0, cb1,
                  acc_sh, sgs0, sgs1, sgs2, sgd0, sgd1, sgd2, ss0, ss1):
    c = lax.axis_index("c")
    s = lax.axis_index("s")
    gs = (gs0, gs1, gs2)
    gd = (gd0, gd1, gd2)
    cb = (cb0, cb1)
    sgs = (sgs0, sgs1, sgs2)
    sgd = (sgd0, sgd1, sgd2)
    ss = (ss0, ss1)
    is_num = c == 0

    pltpu.sync_copy(zeros_hbm.at[pl.ds(s * ZR, ZR)],
                    acc_sh.at[pl.ds(s * ZR, ZR)])
    plsc.subcore_barrier()

    def block(b, carry):
        row0 = s * CHUNKS + b * K
        pltpu.sync_copy(src_hbm.at[pl.ds(row0, K)], sidxb)
        pltpu.sync_copy(dst_hbm.at[pl.ds(row0, K)], didxb)

        hg = [None, None, None]
        hd = [None, None, None]
        hs = [None, None]
        for k0 in range(2):
            hg[k0] = pltpu.async_copy(ts_hbm.at[sidxb.at[k0]], gs[k0], sgs[k0])
            hd[k0] = pltpu.async_copy(td_hbm.at[didxb.at[k0]], gd[k0], sgd[k0])
        for k in range(K):
            sl = k % 3
            cs = k % 2
            if k < K - 2:
                ns_ = (k + 2) % 3
                hg[ns_] = pltpu.async_copy(ts_hbm.at[sidxb.at[k + 2]],
                                           gs[ns_], sgs[ns_])
                hd[ns_] = pltpu.async_copy(td_hbm.at[didxb.at[k + 2]],
                                           gd[ns_], sgd[ns_])
            hg[sl].wait()
            hd[sl].wait()
            if k >= 2:
                hs[cs].wait()

            gsl = gs[sl]
            gdl = gd[sl]
            cbl = cb[cs]

            @pl.when(is_num)
            def _():
                @plsc.parallel_loop(0, C, step=1, unroll=16)
                def _num(r):
                    xwp = gsl[r, pl.ds(0, 16)]
                    asr = gsl[r, pl.ds(16, 16)]
                    adr = gdl[r, pl.ds(0, 16)]
                    e = asr + adr
                    e = jnp.where(e >= 0.0, e, 0.2 * e)
                    cbl[r, pl.ds(0, 16)] = xwp * jnp.exp(e)

            @pl.when(jnp.logical_not(is_num))
            def _():
                @plsc.parallel_loop(0, C, step=1, unroll=16)
                def _den(r):
                    asr = gsl[r, pl.ds(16, 16)]
                    adr = gdl[r, pl.ds(0, 16)]
                    e = asr + adr
                    e = jnp.where(e >= 0.0, e, 0.2 * e)
                    cbl[r, pl.ds(0, 16)] = jnp.exp(e)
            hs[cs] = pltpu.async_copy(cbl, acc_sh.at[didxb.at[k]], ss[cs],
                                      add=True)
        hs[0].wait()
        hs[1].wait()
        return carry

    lax.fori_loop(0, CHUNKS // K, block, 0)
    plsc.subcore_barrier()
    pltpu.sync_copy(acc_sh.at[pl.ds(s * ZR, ZR)],
                    out_hbm.at[pl.ds(c * N_PAD + s * ZR, ZR)])


def _sc_layer(src, dst, ts, td, zeros):
    mesh = plsc.VectorSubcoreMesh(core_axis_name="c", subcore_axis_name="s")
    f = pl.kernel(
        _sc_edge_body,
        mesh=mesh,
        out_type=jax.ShapeDtypeStruct((2 * N_PAD, 16), jnp.float32),
        scratch_types=[
            pltpu.VMEM((K, C), jnp.int32),
            pltpu.VMEM((K, C), jnp.int32),
            pltpu.VMEM((C, 32), jnp.float32),
            pltpu.VMEM((C, 32), jnp.float32),
            pltpu.VMEM((C, 32), jnp.float32),
            pltpu.VMEM((C, 16), jnp.float32),
            pltpu.VMEM((C, 16), jnp.float32),
            pltpu.VMEM((C, 16), jnp.float32),
            pltpu.VMEM((C, 16), jnp.float32),
            pltpu.VMEM((C, 16), jnp.float32),
            pltpu.VMEM_SHARED((N_PAD, 16), jnp.float32),
            pltpu.SemaphoreType.DMA,
            pltpu.SemaphoreType.DMA,
            pltpu.SemaphoreType.DMA,
            pltpu.SemaphoreType.DMA,
            pltpu.SemaphoreType.DMA,
            pltpu.SemaphoreType.DMA,
            pltpu.SemaphoreType.DMA,
            pltpu.SemaphoreType.DMA,
        ],
        compiler_params=pltpu.CompilerParams(use_tc_tiling_on_sc=False),
    )
    return f(src, dst, ts, td, zeros)


def _prep0_body(nt_ref, rq_ref, ew_ref, w16_ref, mts_ref, mtd_ref,
                ts_ref, td_ref):
    nt = nt_ref[...]
    sel = (nt == lax.broadcasted_iota(jnp.int32, (BN, 4), 1))
    xw = jnp.dot(sel.astype(jnp.float32), ew_ref[...],
                 preferred_element_type=jnp.float32)
    xw = xw + rq_ref[...] * w16_ref[...]
    ts_ref[...] = jnp.dot(xw, mts_ref[...], preferred_element_type=jnp.float32)
    td_ref[...] = jnp.dot(xw, mtd_ref[...], preferred_element_type=jnp.float32)


def _prep_body(p0_ref, p1_ref, b_ref, w_ref, mts_ref, mtd_ref,
               ts_ref, td_ref):
    z = jax.nn.relu(p0_ref[...] / p1_ref[...] + b_ref[...])
    xw = jnp.dot(z, w_ref[...], preferred_element_type=jnp.float32)
    ts_ref[...] = jnp.dot(xw, mts_ref[...], preferred_element_type=jnp.float32)
    td_ref[...] = jnp.dot(xw, mtd_ref[...], preferred_element_type=jnp.float32)


def _mlp_body(p0_ref, p1_ref, b_ref, pm_ref,
              w0_ref, b0_ref, w1_ref, b1_ref, w2_ref, b2_ref,
              w3_ref, b3_ref, wo_ref, bo_ref, lg_ref, h_ref):
    yp = p0_ref[...] / p1_ref[...] + b_ref[...]
    h = jnp.dot(yp, pm_ref[...], preferred_element_type=jnp.float32)
    h_ref[...] = h
    y = jax.nn.relu(jnp.dot(h, w0_ref[...],
                            preferred_element_type=jnp.float32) + b0_ref[...])
    for w, b in ((w1_ref, b1_ref), (w2_ref, b2_ref), (w3_ref, b3_ref)):
        y = jax.nn.relu(jnp.dot(y, w[...],
                                preferred_element_type=jnp.float32) + b[...])
    lg_ref[...] = jnp.dot(y, wo_ref[...],
                          preferred_element_type=jnp.float32) + bo_ref[...]


def _matvec_body(h_ref, hp_ref, m_ref, o_ref):
    o_ref[...] = jnp.dot(h_ref[...], hp_ref[...],
                         preferred_element_type=jnp.float32) + m_ref[...]


def _row_spec(w):
    return pl.BlockSpec((BN, w), lambda i: (i, 0))


def _full_spec(a, b):
    return pl.BlockSpec((a, b), lambda i: (0, 0))


def _prep0(nt, rq, ew, w16, mts, mtd):
    return pl.pallas_call(
        _prep0_body, grid=(NBLK,),
        in_specs=[_row_spec(1), _row_spec(1), _full_spec(4, 16),
                  _full_spec(1, 16), _full_spec(16, 32), _full_spec(16, 16)],
        out_specs=[_row_spec(32), _row_spec(16)],
        out_shape=[jax.ShapeDtypeStruct((N_PAD, 32), jnp.float32),
                   jax.ShapeDtypeStruct((N_PAD, 16), jnp.float32)],
    )(nt, rq, ew, w16, mts, mtd)


def _prep(p0, p1, b, w, mts, mtd):
    return pl.pallas_call(
        _prep_body, grid=(NBLK,),
        in_specs=[_row_spec(16), _row_spec(16),
                  _full_spec(1, 16), _full_spec(16, 16),
                  _full_spec(16, 32), _full_spec(16, 16)],
        out_specs=[_row_spec(32), _row_spec(16)],
        out_shape=[jax.ShapeDtypeStruct((N_PAD, 32), jnp.float32),
                   jax.ShapeDtypeStruct((N_PAD, 16), jnp.float32)],
    )(p0, p1, b, w, mts, mtd)


def _mlp(p0, p1, bp, pm, params):
    args = [p0, p1, bp, pm, params['W0'], params['b0'].reshape(1, FC)]
    for w, b in zip(params['Wh'], params['bh']):
        args += [w, b.reshape(1, FC)]
    args += [params['Wo'], params['bo'].reshape(1, 1)]
    return pl.pallas_call(
        _mlp_body, grid=(NBLK,),
        in_specs=[_row_spec(16), _row_spec(16),
                  _full_spec(1, 16), _full_spec(16, 16),
                  _full_spec(HID, FC), _full_spec(1, FC),
                  _full_spec(FC, FC), _full_spec(1, FC),
                  _full_spec(FC, FC), _full_spec(1, FC),
                  _full_spec(FC, FC), _full_spec(1, FC),
                  _full_spec(FC, 1), _full_spec(1, 1)],
        out_specs=[_row_spec(1), _row_spec(16)],
        out_shape=[jax.ShapeDtypeStruct((N_PAD, 1), jnp.float32),
                   jax.ShapeDtypeStruct((N_PAD, 16), jnp.float32)],
    )(*args)


def _matvec(h, hp, m):
    return pl.pallas_call(
        _matvec_body, grid=(NBLK,),
        in_specs=[_row_spec(16), _full_spec(16, 1), _row_spec(1)],
        out_specs=_row_spec(1),
        out_shape=jax.ShapeDtypeStruct((N_PAD, 1), jnp.float32),
    )(h, hp, m)


def _fold(a):
    s = jnp.zeros((HID, HEADS), jnp.float32)
    for h in range(HEADS):
        s = s.at[h * DH:(h + 1) * DH, h].set(a[h])
    return s


def kernel(node_type, requests, edge_index, active_mask, params):
    mean_r = jnp.mean(requests[L:])
    std_r = jnp.std(requests[L:], ddof=1)
    req_final = jnp.concatenate([requests[:L], (requests[L:] - mean_r) / std_r])
    rq = jnp.pad(req_final, (0, N_PAD - N)).reshape(N_PAD, 1)
    nt = jnp.pad(node_type.astype(jnp.int32), (0, N_PAD - N)).reshape(N_PAD, 1)

    loops = jnp.arange(N, dtype=edge_index.dtype)
    src = jnp.concatenate([edge_index[0], loops]).astype(jnp.int32)
    dst = jnp.concatenate([edge_index[1], loops]).astype(jnp.int32)
    src = jnp.pad(src, (0, E_PAD - E_TOT), constant_values=N).reshape(TOT_CH, C)
    dst = jnp.pad(dst, (0, E_PAD - E_TOT), constant_values=N).reshape(TOT_CH, C)

    gat = params['gat']
    pm = jnp.eye(HID, dtype=jnp.float32)[:, _PERM]
    trep = jnp.tile(jnp.eye(HEADS, dtype=jnp.float32), (1, 4))
    mts = [jnp.concatenate([pm, _fold(g['a_s']) @ trep], axis=1) for g in gat]
    mtd = [_fold(g['a_d']) @ trep for g in gat]
    weff = [None] + [pm @ g['W'] for g in gat[1:]]
    bperm = [g['b'] @ pm for g in gat]
    ew0 = params['emb'] @ gat[0]['W'][:FD]
    w16 = gat[0]['W'][FD].reshape(1, HID)
    zeros = jnp.zeros((N_PAD, 16), jnp.float32)

    ts, td = _prep0(nt, rq, ew0, w16, mts[0], mtd[0])

    for li in range(4):
        part = _sc_layer(src, dst, ts, td, zeros)
        p0 = part[:N_PAD]
        p1 = part[N_PAD:]
        if li < 3:
            ts, td = _prep(p0, p1, bperm[li].reshape(1, HID),
                           weff[li + 1], mts[li + 1], mtd[li + 1])

    lg, h = _mlp(p0, p1, bperm[3].reshape(1, HID), pm, params)
    logits1 = lg[:N, 0]
    h = h[:N]

    head = active_mask[:L]
    flipped = jnp.where(head == 0, -jnp.inf,
                        jnp.where(jnp.isneginf(head), 0.0, head))
    remove_mask = jnp.concatenate([flipped, active_mask[L:]])
    removed_logits = logits1 + remove_mask
    g1 = jax.random.gumbel(jax.random.key(1), (N,))
    a1 = jnp.argmax(removed_logits + g1).astype(jnp.int32)
    mask2 = active_mask.at[a1].set(0.0)
    hp = jnp.tanh(h[a1] @ params['Wp'] + params['bp']).reshape(HID, 1)
    m2 = jnp.pad(mask2, (0, N_PAD - N)).reshape(N_PAD, 1)
    new_logits = _matvec(jnp.pad(h, ((0, N_PAD - N), (0, 0))), hp, m2)[:N, 0]
    g2 = jax.random.gumbel(jax.random.key(2), (N,))
    a2 = jnp.argmax(new_logits + g2).astype(jnp.int32)
    logits = jnp.stack([removed_logits, new_logits])
    actions = jnp.stack([a1, a2])
    return logits, actions


# speedup vs baseline: 228.4988x; 1.0659x over previous
import jax
import jax.numpy as jnp
from jax import lax
from jax.experimental import pallas as pl
from jax.experimental.pallas import tpu as pltpu
from jax.experimental.pallas import tpu_sc as plsc

N = 100000
L = 15
FD = 16
HID = 16
HEADS = 4
DH = HID // HEADS
FC = 128

BN = 1024
NBLK = 98
N_PAD = BN * NBLK
NC, NS = 2, 16
C = 128
K = 16
E_TOT = 1600000 + N
CHUNKS = K * (-(-E_TOT // (NS * C * K)))
EPT = CHUNKS * C
E_PAD = NS * EPT
TOT_CH = NS * CHUNKS
ZR = N_PAD // NS

_PERM = [(j % 4) * 4 + j // 4 for j in range(16)]


def _sc_edge_body(src_hbm, dst_hbm, ts_hbm, td_hbm, zeros_hbm, out_hbm,
                  sidxb, didxb, gs0, gs1, gs2, gd0, gd1, gd2, cb0, cb1,
                  acc_sh, sgs0, sgs1, sgs2, sgd0, sgd1, sgd2, ss0, ss1):
    c = lax.axis_index("c")
    s = lax.axis_index("s")
    gs = (gs0, gs1, gs2)
    gd = (gd0, gd1, gd2)
    cb = (cb0, cb1)
    sgs = (sgs0, sgs1, sgs2)
    sgd = (sgd0, sgd1, sgd2)
    ss = (ss0, ss1)
    is_num = c == 0

    pltpu.sync_copy(zeros_hbm.at[pl.ds(s * ZR, ZR)],
                    acc_sh.at[pl.ds(s * ZR, ZR)])
    plsc.subcore_barrier()

    def block(b, carry):
        row0 = s * CHUNKS + b * K
        pltpu.sync_copy(src_hbm.at[pl.ds(row0, K)], sidxb)
        pltpu.sync_copy(dst_hbm.at[pl.ds(row0, K)], didxb)

        hg = [None, None, None]
        hd = [None, None, None]
        hs = [None, None]
        for k0 in range(2):
            hg[k0] = pltpu.async_copy(ts_hbm.at[sidxb.at[k0]], gs[k0], sgs[k0])
            hd[k0] = pltpu.async_copy(td_hbm.at[didxb.at[k0]], gd[k0], sgd[k0])
        for k in range(K):
            sl = k % 3
            cs = k % 2
            if k < K - 2:
                ns_ = (k + 2) % 3
                hg[ns_] = pltpu.async_copy(ts_hbm.at[sidxb.at[k + 2]],
                                           gs[ns_], sgs[ns_])
                hd[ns_] = pltpu.async_copy(td_hbm.at[didxb.at[k + 2]],
                                           gd[ns_], sgd[ns_])
            hg[sl].wait()
            hd[sl].wait()
            if k >= 2:
                hs[cs].wait()

            gsl = gs[sl]
            gdl = gd[sl]
            cbl = cb[cs]

            @pl.when(is_num)
            def _():
                @plsc.parallel_loop(0, C, step=1, unroll=16)
                def _num(r):
                    xwp = gsl[r, pl.ds(0, 16)]
                    asr = gsl[r, pl.ds(16, 16)]
                    adr = gdl[r, pl.ds(0, 16)]
                    e = asr + adr
                    e = jnp.where(e >= 0.0, e, 0.2 * e)
                    cbl[r, pl.ds(0, 16)] = xwp * jnp.exp(e)

            @pl.when(jnp.logical_not(is_num))
            def _():
                @plsc.parallel_loop(0, C, step=1, unroll=16)
                def _den(r):
                    asr = gsl[r, pl.ds(16, 16)]
                    adr = gdl[r, pl.ds(0, 16)]
                    e = asr + adr
                    e = jnp.where(e >= 0.0, e, 0.2 * e)
                    cbl[r, pl.ds(0, 16)] = jnp.exp(e)
            hs[cs] = pltpu.async_copy(cbl, acc_sh.at[didxb.at[k]], ss[cs],
                                      add=True)
        hs[0].wait()
        hs[1].wait()
        return carry

    lax.fori_loop(0, CHUNKS // K, block, 0)
    plsc.subcore_barrier()
    pltpu.sync_copy(acc_sh.at[pl.ds(s * ZR, ZR)],
                    out_hbm.at[pl.ds(c * N_PAD + s * ZR, ZR)])


def _sc_layer(src, dst, ts, td, zeros):
    mesh = plsc.VectorSubcoreMesh(core_axis_name="c", subcore_axis_name="s")
    f = pl.kernel(
        _sc_edge_body,
        mesh=mesh,
        out_type=jax.ShapeDtypeStruct((2 * N_PAD, 16), jnp.float32),
        scratch_types=[
            pltpu.VMEM((K, C), jnp.int32),
            pltpu.VMEM((K, C), jnp.int32),
            pltpu.VMEM((C, 32), jnp.float32),
            pltpu.VMEM((C, 32), jnp.float32),
            pltpu.VMEM((C, 32), jnp.float32),
            pltpu.VMEM((C, 16), jnp.float32),
            pltpu.VMEM((C, 16), jnp.float32),
            pltpu.VMEM((C, 16), jnp.float32),
            pltpu.VMEM((C, 16), jnp.float32),
            pltpu.VMEM((C, 16), jnp.float32),
            pltpu.VMEM_SHARED((N_PAD, 16), jnp.float32),
            pltpu.SemaphoreType.DMA,
            pltpu.SemaphoreType.DMA,
            pltpu.SemaphoreType.DMA,
            pltpu.SemaphoreType.DMA,
            pltpu.SemaphoreType.DMA,
            pltpu.SemaphoreType.DMA,
            pltpu.SemaphoreType.DMA,
            pltpu.SemaphoreType.DMA,
        ],
        compiler_params=pltpu.CompilerParams(use_tc_tiling_on_sc=False),
    )
    return f(src, dst, ts, td, zeros)


def _prep0_body(nt_ref, rq_ref, ew_ref, w16_ref, mts_ref, mtd_ref,
                ts_ref, td_ref):
    nt = nt_ref[...]
    sel = (nt == lax.broadcasted_iota(jnp.int32, (BN, 4), 1))
    xw = jnp.dot(sel.astype(jnp.float32), ew_ref[...],
                 preferred_element_type=jnp.float32)
    xw = xw + rq_ref[...] * w16_ref[...]
    ts_ref[...] = jnp.dot(xw, mts_ref[...], preferred_element_type=jnp.float32)
    td_ref[...] = jnp.dot(xw, mtd_ref[...], preferred_element_type=jnp.float32)


def _prep_body(p0_ref, p1_ref, b_ref, w_ref, mts_ref, mtd_ref,
               ts_ref, td_ref):
    z = jax.nn.relu(p0_ref[...] / p1_ref[...] + b_ref[...])
    xw = jnp.dot(z, w_ref[...], preferred_element_type=jnp.float32)
    ts_ref[...] = jnp.dot(xw, mts_ref[...], preferred_element_type=jnp.float32)
    td_ref[...] = jnp.dot(xw, mtd_ref[...], preferred_element_type=jnp.float32)


def _mlp_body(p0_ref, p1_ref, b_ref, pm_ref,
              w0_ref, b0_ref, w1_ref, b1_ref, w2_ref, b2_ref,
              w3_ref, b3_ref, wo_ref, bo_ref, lg_ref, h_ref):
    yp = p0_ref[...] / p1_ref[...] + b_ref[...]
    h = jnp.dot(yp, pm_ref[...], preferred_element_type=jnp.float32)
    h_ref[...] = h
    y = jax.nn.relu(jnp.dot(h, w0_ref[...],
                            preferred_element_type=jnp.float32) + b0_ref[...])
    for w, b in ((w1_ref, b1_ref), (w2_ref, b2_ref), (w3_ref, b3_ref)):
        y = jax.nn.relu(jnp.dot(y, w[...],
                                preferred_element_type=jnp.float32) + b[...])
    lg_ref[...] = jnp.dot(y, wo_ref[...],
                          preferred_element_type=jnp.float32) + bo_ref[...]


def _matvec_body(h_ref, hp_ref, m_ref, o_ref):
    o_ref[...] = jnp.dot(h_ref[...], hp_ref[...],
                         preferred_element_type=jnp.float32) + m_ref[...]


def _row_spec(w):
    return pl.BlockSpec((BN, w), lambda i: (i, 0))


def _full_spec(a, b):
    return pl.BlockSpec((a, b), lambda i: (0, 0))


def _prep0(nt, rq, ew, w16, mts, mtd):
    return pl.pallas_call(
        _prep0_body, grid=(NBLK,),
        in_specs=[_row_spec(1), _row_spec(1), _full_spec(4, 16),
                  _full_spec(1, 16), _full_spec(16, 32), _full_spec(16, 16)],
        out_specs=[_row_spec(32), _row_spec(16)],
        out_shape=[jax.ShapeDtypeStruct((N_PAD, 32), jnp.float32),
                   jax.ShapeDtypeStruct((N_PAD, 16), jnp.float32)],
    )(nt, rq, ew, w16, mts, mtd)


def _prep(p0, p1, b, w, mts, mtd):
    return pl.pallas_call(
        _prep_body, grid=(NBLK,),
        in_specs=[_row_spec(16), _row_spec(16),
                  _full_spec(1, 16), _full_spec(16, 16),
                  _full_spec(16, 32), _full_spec(16, 16)],
        out_specs=[_row_spec(32), _row_spec(16)],
        out_shape=[jax.ShapeDtypeStruct((N_PAD, 32), jnp.float32),
                   jax.ShapeDtypeStruct((N_PAD, 16), jnp.float32)],
    )(p0, p1, b, w, mts, mtd)


def _mlp(p0, p1, bp, pm, params):
    args = [p0, p1, bp, pm, params['W0'], params['b0'].reshape(1, FC)]
    for w, b in zip(params['Wh'], params['bh']):
        args += [w, b.reshape(1, FC)]
    args += [params['Wo'], params['bo'].reshape(1, 1)]
    return pl.pallas_call(
        _mlp_body, grid=(NBLK,),
        in_specs=[_row_spec(16), _row_spec(16),
                  _full_spec(1, 16), _full_spec(16, 16),
                  _full_spec(HID, FC), _full_spec(1, FC),
                  _full_spec(FC, FC), _full_spec(1, FC),
                  _full_spec(FC, FC), _full_spec(1, FC),
                  _full_spec(FC, FC), _full_spec(1, FC),
                  _full_spec(FC, 1), _full_spec(1, 1)],
        out_specs=[_row_spec(1), _row_spec(16)],
        out_shape=[jax.ShapeDtypeStruct((N_PAD, 1), jnp.float32),
                   jax.ShapeDtypeStruct((N_PAD, 16), jnp.float32)],
    )(*args)


def _matvec(h, hp, m):
    return pl.pallas_call(
        _matvec_body, grid=(NBLK,),
        in_specs=[_row_spec(16), _full_spec(16, 1), _row_spec(1)],
        out_specs=_row_spec(1),
        out_shape=jax.ShapeDtypeStruct((N_PAD, 1), jnp.float32),
    )(h, hp, m)


def _fold(a):
    s = jnp.zeros((HID, HEADS), jnp.float32)
    for h in range(HEADS):
        s = s.at[h * DH:(h + 1) * DH, h].set(a[h])
    return s


def kernel(node_type, requests, edge_index, active_mask, params):
    mean_r = jnp.mean(requests[L:])
    std_r = jnp.std(requests[L:], ddof=1)
    req_final = jnp.concatenate([requests[:L], (requests[L:] - mean_r) / std_r])
    rq = jnp.pad(req_final, (0, N_PAD - N)).reshape(N_PAD, 1)
    nt = jnp.pad(node_type.astype(jnp.int32), (0, N_PAD - N)).reshape(N_PAD, 1)

    loops = jnp.arange(N, dtype=edge_index.dtype)
    src = jnp.concatenate([edge_index[0], loops]).astype(jnp.int32)
    dst = jnp.concatenate([edge_index[1], loops]).astype(jnp.int32)
    src = jnp.pad(src, (0, E_PAD - E_TOT), constant_values=N).reshape(TOT_CH, C)
    dst = jnp.pad(dst, (0, E_PAD - E_TOT), constant_values=N).reshape(TOT_CH, C)

    gat = params['gat']
    pm = jnp.eye(HID, dtype=jnp.float32)[:, _PERM]
    trep = jnp.tile(jnp.eye(HEADS, dtype=jnp.float32), (1, 4))
    mts = [jnp.concatenate([pm, _fold(g['a_s']) @ trep], axis=1) for g in gat]
    mtd = [_fold(g['a_d']) @ trep for g in gat]
    weff = [None] + [pm @ g['W'] for g in gat[1:]]
    bperm = [g['b'] @ pm for g in gat]
    ew0 = params['emb'] @ gat[0]['W'][:FD]
    w16 = gat[0]['W'][FD].reshape(1, HID)
    zeros = jnp.zeros((N_PAD, 16), jnp.float32)

    ts, td = _prep0(nt, rq, ew0, w16, mts[0], mtd[0])

    for li in range(4):
        part = _sc_layer(src, dst, ts, td, zeros)
        p0 = part[:N_PAD]
        p1 = part[N_PAD:]
        if li < 3:
            ts, td = _prep(p0, p1, bperm[li].reshape(1, HID),
                           weff[li + 1], mts[li + 1], mtd[li + 1])

    lg, h = _mlp(p0, p1, bperm[3].reshape(1, HID), pm, params)
    logits1 = lg[:N, 0]
    h = h[:N]

    head = active_mask[:L]
    flipped = jnp.where(head == 0, -jnp.inf,
                        jnp.where(jnp.isneginf(head), 0.0, head))
    remove_mask = jnp.concatenate([flipped, active_mask[L:]])
    removed_logits = logits1 + remove_mask
    g1 = jax.random.gumbel(jax.random.key(1), (N,))
    a1 = jnp.argmax(removed_logits + g1).astype(jnp.int32)
    mask2 = active_mask.at[a1].set(0.0)
    hp = jnp.tanh(h[a1] @ params['Wp'] + params['bp']).reshape(HID, 1)
    m2 = jnp.pad(mask2, (0, N_PAD - N)).reshape(N_PAD, 1)
    new_logits = _matvec(jnp.pad(h, ((0, N_PAD - N), (0, 0))), hp, m2)[:N, 0]
    g2 = jax.random.gumbel(jax.random.key(2), (N,))
    a2 = jnp.argmax(new_logits + g2).astype(jnp.int32)
    logits = jnp.stack([removed_logits, new_logits])
    actions = jnp.stack([a1, a2])
    return logits, actions
